# baseline (speedup 1.0000x reference)
import functools

import jax
import jax.numpy as jnp
import numpy as np
from jax import lax
from jax.experimental import pallas as pl
from jax.experimental.pallas import tpu as pltpu
from jax.experimental.pallas import tpu_sc as plsc

B = 8
C = 3
N = 16384
K = 8
L = 8
M = 8

NW = 32
WPB = NW // B
NPW = N // WPB
SUB = 512
NB = 2048

LOG2E = 1.4426950408889634
NEG2LOG2E = -2.0 * LOG2E


CH = 2048


def _knn_t_body(k_ref, out_ref):
    out_ref[0] = k_ref[0].T


def _knn_t(knn):
    return pl.pallas_call(
        _knn_t_body,
        grid=(B, N // CH),
        in_specs=[pl.BlockSpec((1, CH, K), lambda b, c: (b, c, 0))],
        out_specs=pl.BlockSpec((1, K, CH), lambda b, c: (b, 0, c)),
        out_shape=jax.ShapeDtypeStruct((B, K, N), jnp.int32),
    )(knn)


def _sc_gather_body(points_hbm, knn_hbm, xc_hbm, tab_v, knn_v, out_v):
    wid = lax.axis_index("s") * 2 + lax.axis_index("c")
    b = wid // WPB
    n0 = (wid % WPB) * NPW

    pltpu.sync_copy(points_hbm.at[b, pl.ds(0, C)], tab_v)
    pltpu.sync_copy(knn_hbm.at[b, :, pl.ds(n0, NPW)], knn_v)

    c0 = jnp.zeros((16,), jnp.int32)
    c1 = jnp.full((16,), 1, jnp.int32)
    c2v = jnp.full((16,), 2, jnp.int32)
    ones_f = jnp.full((16,), 1.0, jnp.float32)
    zero_f = jnp.zeros((16,), jnp.float32)

    def init_rows(i, _):
        for k in range(K):
            out_v[k, 4, pl.ds(i * 16, 16)] = ones_f
            for r in range(5, 8):
                out_v[k, r, pl.ds(i * 16, 16)] = zero_f
        return 0

    lax.fori_loop(0, SUB // 16, init_rows, 0)

    def group(g, s_base):
        local = s_base + g * 16
        gbase = n0 + local
        cx = tab_v[0, pl.ds(gbase, 16)]
        cy = tab_v[1, pl.ds(gbase, 16)]
        cz = tab_v[2, pl.ds(gbase, 16)]
        off = g * 16
        for k in range(K):
            idx = knn_v[k, pl.ds(local, 16)]
            vx = plsc.load_gather(tab_v, [c0, idx]) - cx
            vy = plsc.load_gather(tab_v, [c1, idx]) - cy
            vz = plsc.load_gather(tab_v, [c2v, idx]) - cz
            out_v[k, 0, pl.ds(off, 16)] = vx
            out_v[k, 1, pl.ds(off, 16)] = vy
            out_v[k, 2, pl.ds(off, 16)] = vz
            out_v[k, 3, pl.ds(off, 16)] = (
                vx * vx + vy * vy + vz * vz) * NEG2LOG2E

    for s in range(NPW // SUB):
        s_base = s * SUB
        lax.fori_loop(0, SUB // 16, lambda g, _: (group(g, s_base), 0)[1], 0)
        pltpu.sync_copy(out_v, xc_hbm.at[b, :, :, pl.ds(n0 + s_base, SUB)])


_SC_GATHER_CACHE = []


def _sc_gather(points, knn_flat):
    if not _SC_GATHER_CACHE:
        _SC_GATHER_CACHE.append(functools.partial(
            pl.kernel,
            out_type=jax.ShapeDtypeStruct((B, K, 8, N), jnp.float32),
            mesh=plsc.VectorSubcoreMesh(core_axis_name="c", subcore_axis_name="s"),
            scratch_types=[
                pltpu.VMEM((C, N), jnp.float32),
                pltpu.VMEM((K, NPW), jnp.int32),
                pltpu.VMEM((K, 8, SUB), jnp.float32),
            ],
            compiler_params=pltpu.CompilerParams(needs_layout_passes=False),
        )(_sc_gather_body))
    return _SC_GATHER_CACHE[0](points, knn_flat)


def _tc_body(w_ref, sel_ref, xc_ref, out_ref):
    acc = None
    for k in range(K):
        xk = xc_ref[0, k]
        t = xk[3:4, :]
        m = jax.lax.dot_general(
            w_ref[...], xk.astype(jnp.bfloat16),
            (((1,), (0,)), ((), ())),
            preferred_element_type=jnp.float32)
        e = jnp.exp2(m + t)
        acc = e if acc is None else acc + e
    out_ref[0] = jax.lax.dot_general(
        sel_ref[...], acc, (((1,), (0,)), ((), ())),
        preferred_element_type=jnp.float32)


def _tc_compute(w, sel, xc):
    return pl.pallas_call(
        _tc_body,
        grid=(B, N // NB),
        in_specs=[
            pl.BlockSpec((L * M, 8), lambda b, n: (0, 0)),
            pl.BlockSpec((L, L * M), lambda b, n: (0, 0)),
            pl.BlockSpec((1, K, 8, NB), lambda b, n: (b, 0, 0, n)),
        ],
        out_specs=pl.BlockSpec((1, L, NB), lambda b, n: (b, 0, n)),
        out_shape=jax.ShapeDtypeStruct((B, L, N), jnp.float32),
    )(w, sel, xc)


_SEL = np.asarray(np.kron(np.eye(L), np.ones((1, M))) / K, np.float32)


def kernel(points, knn_graph, kernel):
    points8 = jnp.pad(points, ((0, 0), (0, 8 - C), (0, 0)))
    xc = _sc_gather(points8, _knn_t(knn_graph))
    kf = kernel.reshape(L * M, C)
    k2 = jnp.sum(kf * kf, axis=1)
    w = jnp.concatenate([
        (4.0 * LOG2E) * kf,
        jnp.zeros((L * M, 1), jnp.float32),
        (NEG2LOG2E * k2)[:, None],
        jnp.zeros((L * M, 3), jnp.float32),
    ], axis=1).astype(jnp.bfloat16)
    return _tc_compute(w, jnp.asarray(_SEL), xc)


# speedup vs baseline: 1.2119x; 1.2119x over previous
import functools

import jax
import jax.numpy as jnp
import numpy as np
from jax import lax
from jax.experimental import pallas as pl
from jax.experimental.pallas import tpu as pltpu
from jax.experimental.pallas import tpu_sc as plsc

B = 8
C = 3
N = 16384
K = 8
L = 8
M = 8

NCHUNK = 2
HALF = N // NCHUNK

NW = 32
WPB = NW // B
NPW = HALF // WPB
SUB = 512
NB = 2048
CH = 2048

LOG2E = 1.4426950408889634
NEG2LOG2E = -2.0 * LOG2E


def _knn_t_body(k_ref, out_ref):
    out_ref[0] = k_ref[0].T


def _knn_t(knn, off):
    return pl.pallas_call(
        _knn_t_body,
        grid=(B, HALF // CH),
        in_specs=[pl.BlockSpec((1, CH, K), lambda b, c: (b, off // CH + c, 0))],
        out_specs=pl.BlockSpec((1, K, CH), lambda b, c: (b, 0, c)),
        out_shape=jax.ShapeDtypeStruct((B, K, HALF), jnp.int32),
    )(knn)


def _make_sc_body(off):
    def _sc_gather_body(points_hbm, knn_hbm, xc_hbm, tab_v, knn_v, out_v):
        wid = lax.axis_index("s") * 2 + lax.axis_index("c")
        b = wid // WPB
        n0 = (wid % WPB) * NPW

        pltpu.sync_copy(points_hbm.at[b], tab_v)
        pltpu.sync_copy(knn_hbm.at[b, :, pl.ds(n0, NPW)], knn_v)

        c0 = jnp.zeros((16,), jnp.int32)
        c1 = jnp.full((16,), 1, jnp.int32)
        c2v = jnp.full((16,), 2, jnp.int32)
        ones_f = jnp.full((16,), 1.0, jnp.float32)
        zero_f = jnp.zeros((16,), jnp.float32)

        def init_rows(i, _):
            for k in range(K):
                out_v[k, 4, pl.ds(i * 16, 16)] = ones_f
                for r in range(5, 8):
                    out_v[k, r, pl.ds(i * 16, 16)] = zero_f
            return 0

        lax.fori_loop(0, SUB // 16, init_rows, 0)

        def group(g, s_base):
            local = s_base + g * 16
            gbase = off + n0 + local
            cx = tab_v[0, pl.ds(gbase, 16)]
            cy = tab_v[1, pl.ds(gbase, 16)]
            cz = tab_v[2, pl.ds(gbase, 16)]
            offo = g * 16
            for k in range(K):
                idx = knn_v[k, pl.ds(local, 16)]
                vx = plsc.load_gather(tab_v, [c0, idx]) - cx
                vy = plsc.load_gather(tab_v, [c1, idx]) - cy
                vz = plsc.load_gather(tab_v, [c2v, idx]) - cz
                out_v[k, 0, pl.ds(offo, 16)] = vx
                out_v[k, 1, pl.ds(offo, 16)] = vy
                out_v[k, 2, pl.ds(offo, 16)] = vz
                out_v[k, 3, pl.ds(offo, 16)] = (
                    vx * vx + vy * vy + vz * vz) * NEG2LOG2E

        for s in range(NPW // SUB):
            s_base = s * SUB
            lax.fori_loop(0, SUB // 16, lambda g, _: (group(g, s_base), 0)[1], 0)
            pltpu.sync_copy(out_v, xc_hbm.at[b, :, :, pl.ds(n0 + s_base, SUB)])

    return _sc_gather_body


_SC_GATHER_CACHE = {}


def _sc_gather(points, knn_t, off):
    if off not in _SC_GATHER_CACHE:
        _SC_GATHER_CACHE[off] = functools.partial(
            pl.kernel,
            out_type=jax.ShapeDtypeStruct((B, K, 8, HALF), jnp.float32),
            mesh=plsc.VectorSubcoreMesh(core_axis_name="c", subcore_axis_name="s"),
            scratch_types=[
                pltpu.VMEM((C, N), jnp.float32),
                pltpu.VMEM((K, NPW), jnp.int32),
                pltpu.VMEM((K, 8, SUB), jnp.float32),
            ],
            compiler_params=pltpu.CompilerParams(needs_layout_passes=False),
        )(_make_sc_body(off))
    return _SC_GATHER_CACHE[off](points, knn_t)


def _tc_body(w_ref, sel_ref, xc_ref, out_ref):
    acc = None
    for k in range(K):
        xk = xc_ref[0, k]
        t = xk[3:4, :]
        m = jax.lax.dot_general(
            w_ref[...], xk.astype(jnp.bfloat16),
            (((1,), (0,)), ((), ())),
            preferred_element_type=jnp.float32)
        e = jnp.exp2(m + t)
        acc = e if acc is None else acc + e
    out_ref[0] = jax.lax.dot_general(
        sel_ref[...], acc, (((1,), (0,)), ((), ())),
        preferred_element_type=jnp.float32)


def _tc_compute(w, sel, xc):
    return pl.pallas_call(
        _tc_body,
        grid=(B, HALF // NB),
        in_specs=[
            pl.BlockSpec((L * M, 8), lambda b, n: (0, 0)),
            pl.BlockSpec((L, L * M), lambda b, n: (0, 0)),
            pl.BlockSpec((1, K, 8, NB), lambda b, n: (b, 0, 0, n)),
        ],
        out_specs=pl.BlockSpec((1, L, NB), lambda b, n: (b, 0, n)),
        out_shape=jax.ShapeDtypeStruct((B, L, HALF), jnp.float32),
    )(w, sel, xc)


_SEL = np.asarray(np.kron(np.eye(L), np.ones((1, M))) / K, np.float32)


def kernel(points, knn_graph, kernel):
    kf = kernel.reshape(L * M, C)
    k2 = jnp.sum(kf * kf, axis=1)
    w = jnp.concatenate([
        (4.0 * LOG2E) * kf,
        jnp.zeros((L * M, 1), jnp.float32),
        (NEG2LOG2E * k2)[:, None],
        jnp.zeros((L * M, 3), jnp.float32),
    ], axis=1).astype(jnp.bfloat16)
    sel = jnp.asarray(_SEL)

    outs = []
    for ci in range(NCHUNK):
        off = ci * HALF
        kt = _knn_t(knn_graph, off)
        xc = _sc_gather(points, kt, off)
        outs.append(_tc_compute(w, sel, xc))
    return jnp.concatenate(outs, axis=2)
